# baseline (speedup 1.0000x reference)
import functools

import jax
import jax.numpy as jnp
from jax import lax
from jax.experimental import pallas as pl
from jax.experimental.pallas import tpu as pltpu
from jax.experimental.pallas import tpu_sc as plsc

N_TOKENS = 1000000
N_HIDDEN = 64
BATCH = 16384
N_NEG = 5
TOTAL = BATCH * (1 + N_NEG)
NC = 2
NS = 16
NW = NC * NS
PER_W = TOTAL // NW
CHUNK = 512
N_CHUNKS = PER_W // CHUNK

_SC_PARAMS = pltpu.CompilerParams(
    needs_layout_passes=False, use_tc_tiling_on_sc=False)


def _sc_gather_u_body(u_idx_hbm, u_table_hbm, out_hbm,
                      idx_v, rows_a, rows_b, sem_a, sem_b):
    wid = lax.axis_index("s") * NC + lax.axis_index("c")

    def chunk_body(c, _):
        base = wid * PER_W + c * CHUNK
        pltpu.sync_copy(u_idx_hbm.at[pl.ds(base, CHUNK)], idx_v)
        cp = pltpu.async_copy(u_table_hbm.at[idx_v], rows_a, sem_a)
        cp.wait()
        pltpu.sync_copy(rows_a, out_hbm.at[pl.ds(base, CHUNK)])
        return 0

    lax.fori_loop(0, N_CHUNKS, chunk_body, 0)


_sc_gather_u = functools.partial(
    pl.kernel,
    mesh=plsc.VectorSubcoreMesh(core_axis_name="c", subcore_axis_name="s"),
    out_type=jax.ShapeDtypeStruct((TOTAL, N_HIDDEN), jnp.float32),
    scratch_types=[
        pltpu.VMEM((CHUNK,), jnp.int32),
        pltpu.VMEM((CHUNK, N_HIDDEN), jnp.float32),
        pltpu.VMEM((CHUNK, N_HIDDEN), jnp.float32),
        pltpu.SemaphoreType.DMA,
        pltpu.SemaphoreType.DMA,
    ],
    compiler_params=_SC_PARAMS,
)(_sc_gather_u_body)


def _sc_v_dots_body(v_idx_hbm, v_table_hbm, u_gath_hbm, out_hbm,
                    idx_v, v_rows, u_rows, dots_v, sem_v):
    wid = lax.axis_index("s") * NC + lax.axis_index("c")
    lane = lax.iota(jnp.int32, 16)

    def chunk_body(c, _):
        base = wid * PER_W + c * CHUNK
        pltpu.sync_copy(v_idx_hbm.at[pl.ds(base, CHUNK)], idx_v)
        cp_v = pltpu.async_copy(v_table_hbm.at[idx_v], v_rows, sem_v)
        pltpu.sync_copy(u_gath_hbm.at[pl.ds(base, CHUNK)], u_rows)
        cp_v.wait()

        def group_body(g, _):
            rows = g * 16 + lane

            def d_body(d, acc):
                for k in range(4):
                    dk = d * 4 + k
                    cols = jnp.full((16,), dk, jnp.int32)
                    uu = plsc.load_gather(u_rows, [rows, cols])
                    vv = plsc.load_gather(v_rows, [rows, cols])
                    acc = acc + uu * vv
                return acc

            acc = lax.fori_loop(0, N_HIDDEN // 4, d_body,
                                jnp.zeros((16,), jnp.float32))
            dots_v[pl.ds(g * 16, 16)] = acc
            return 0

        lax.fori_loop(0, CHUNK // 16, group_body, 0)
        pltpu.sync_copy(dots_v, out_hbm.at[pl.ds(base, CHUNK)])
        return 0

    lax.fori_loop(0, N_CHUNKS, chunk_body, 0)


_sc_v_dots = functools.partial(
    pl.kernel,
    mesh=plsc.VectorSubcoreMesh(core_axis_name="c", subcore_axis_name="s"),
    out_type=jax.ShapeDtypeStruct((TOTAL,), jnp.float32),
    scratch_types=[
        pltpu.VMEM((CHUNK,), jnp.int32),
        pltpu.VMEM((CHUNK, N_HIDDEN), jnp.float32),
        pltpu.VMEM((CHUNK, N_HIDDEN), jnp.float32),
        pltpu.VMEM((CHUNK,), jnp.float32),
        pltpu.SemaphoreType.DMA,
    ],
    compiler_params=_SC_PARAMS,
)(_sc_v_dots_body)


_ROWS = TOTAL // 128
_POS_ROWS = BATCH // 128


def _tc_loss_body(dots_ref, out_ref):
    x = dots_ref[...]
    row = lax.broadcasted_iota(jnp.int32, (_ROWS, 128), 0)
    sgn = jnp.where(row < _POS_ROWS, 1.0, -1.0).astype(jnp.float32)
    z = x * sgn
    ls = jnp.minimum(z, 0.0) - jnp.log1p(jnp.exp(-jnp.abs(z)))
    out_ref[0, 0] = -jnp.sum(ls)


def kernel(u_pos, v_pos, u_neg, v_neg, u_table, v_table):
    u_idx = jnp.concatenate(
        [u_pos.astype(jnp.int32), u_neg.reshape(-1).astype(jnp.int32)])
    v_idx = jnp.concatenate(
        [v_pos.astype(jnp.int32), v_neg.reshape(-1).astype(jnp.int32)])
    u_gath = _sc_gather_u(u_idx, u_table)
    dots = _sc_v_dots(v_idx, v_table, u_gath)
    loss = pl.pallas_call(
        _tc_loss_body,
        out_shape=jax.ShapeDtypeStruct((1, 1), jnp.float32),
        out_specs=pl.BlockSpec(memory_space=pltpu.SMEM),
    )(dots.reshape(_ROWS, 128))
    return loss[0, 0]


# speedup vs baseline: 1.0169x; 1.0169x over previous
import functools

import jax
import jax.numpy as jnp
from jax import lax
from jax.experimental import pallas as pl
from jax.experimental.pallas import tpu as pltpu
from jax.experimental.pallas import tpu_sc as plsc

N_TOKENS = 1000000
N_HIDDEN = 64
PAIR_W = 128
BATCH = 16384
N_NEG = 5
TOTAL = BATCH * (1 + N_NEG)
NC = 2
NS = 16
NW = NC * NS
PER_W = TOTAL // NW
CHUNK = 128
N_CHUNKS = PER_W // CHUNK

_SC_PARAMS = pltpu.CompilerParams(
    needs_layout_passes=False, use_tc_tiling_on_sc=False)
_MESH = plsc.VectorSubcoreMesh(core_axis_name="c", subcore_axis_name="s")


def _halve_indices(idx_raw, idx_row):
    def body(i, _):
        v = idx_raw[pl.ds(i * 16, 16)]
        idx_row[pl.ds(i * 16, 16)] = lax.shift_right_logical(v, 1)
        return 0
    lax.fori_loop(0, PER_W // 16, body, 0)


def _sc_gather_u_body(u_idx_hbm, u_table_hbm, out_hbm, idx_raw, idx_row,
                      rows, sem_a, sem_b):
    wid = lax.axis_index("s") * NC + lax.axis_index("c")
    base_w = wid * PER_W
    pltpu.sync_copy(u_idx_hbm.at[pl.ds(base_w, PER_W)], idx_raw)
    _halve_indices(idx_raw, idx_row)

    def start(c, par, sem):
        pltpu.async_copy(
            u_table_hbm.at[idx_row.at[pl.ds(c * CHUNK, CHUNK)]],
            rows.at[pl.ds(par * CHUNK, CHUNK)], sem)

    def wait(par, sem):
        pltpu.make_async_copy(
            u_table_hbm.at[idx_row.at[pl.ds(0, CHUNK)]],
            rows.at[pl.ds(par * CHUNK, CHUNK)], sem).wait()

    def writeback(c, par):
        pltpu.sync_copy(
            rows.at[pl.ds(par * CHUNK, CHUNK)],
            out_hbm.at[pl.ds(base_w + c * CHUNK, CHUNK)])

    start(0, 0, sem_a)

    def chunk_body(c2, _):
        c = c2 * 2

        @pl.when(c + 1 < N_CHUNKS)
        def _():
            start(c + 1, 1, sem_b)

        wait(0, sem_a)
        writeback(c, 0)

        @pl.when(c + 2 < N_CHUNKS)
        def _():
            start(c + 2, 0, sem_a)

        @pl.when(c + 1 < N_CHUNKS)
        def _():
            wait(1, sem_b)
            writeback(c + 1, 1)

        return 0

    lax.fori_loop(0, (N_CHUNKS + 1) // 2, chunk_body, 0)


_sc_gather_u = functools.partial(
    pl.kernel,
    mesh=_MESH,
    out_type=jax.ShapeDtypeStruct((TOTAL, PAIR_W), jnp.float32),
    scratch_types=[
        pltpu.VMEM((PER_W,), jnp.int32),
        pltpu.VMEM((PER_W,), jnp.int32),
        pltpu.VMEM((2 * CHUNK, PAIR_W), jnp.float32),
        pltpu.SemaphoreType.DMA,
        pltpu.SemaphoreType.DMA,
    ],
    compiler_params=_SC_PARAMS,
)(_sc_gather_u_body)


def _sc_v_dots_body(v_idx_hbm, u_idx_hbm, v_table_hbm, u_gath_hbm, out_hbm,
                    idx_raw, idx_row, idx_u, v_rows, u_rows, dots_v,
                    sem_a, sem_b, sem_ua, sem_ub):
    wid = lax.axis_index("s") * NC + lax.axis_index("c")
    base_w = wid * PER_W
    lane = lax.iota(jnp.int32, 16)
    pltpu.sync_copy(v_idx_hbm.at[pl.ds(base_w, PER_W)], idx_raw)
    pltpu.sync_copy(u_idx_hbm.at[pl.ds(base_w, PER_W)], idx_u)
    _halve_indices(idx_raw, idx_row)

    def start(c, par, sem, sem_u):
        pltpu.async_copy(
            v_table_hbm.at[idx_row.at[pl.ds(c * CHUNK, CHUNK)]],
            v_rows.at[pl.ds(par * CHUNK, CHUNK)], sem)
        pltpu.async_copy(
            u_gath_hbm.at[pl.ds(base_w + c * CHUNK, CHUNK)],
            u_rows.at[pl.ds(par * CHUNK, CHUNK)], sem_u)

    def wait(par, sem, sem_u):
        pltpu.make_async_copy(
            v_table_hbm.at[idx_row.at[pl.ds(0, CHUNK)]],
            v_rows.at[pl.ds(par * CHUNK, CHUNK)], sem).wait()
        pltpu.make_async_copy(
            u_gath_hbm.at[pl.ds(base_w, CHUNK)],
            u_rows.at[pl.ds(par * CHUNK, CHUNK)], sem_u).wait()

    def compute(c, par):
        def group_body(g, _):
            rows = par * CHUNK + g * 16 + lane
            off_u = (idx_u[pl.ds(c * CHUNK + g * 16, 16)] & 1) * N_HIDDEN
            off_v = (idx_raw[pl.ds(c * CHUNK + g * 16, 16)] & 1) * N_HIDDEN
            acc = [jnp.zeros((16,), jnp.float32) for _ in range(4)]
            for d in range(N_HIDDEN):
                uu = plsc.load_gather(u_rows, [rows, off_u + d])
                vv = plsc.load_gather(v_rows, [rows, off_v + d])
                acc[d % 4] = acc[d % 4] + uu * vv
            dots_v[pl.ds(g * 16, 16)] = (
                (acc[0] + acc[1]) + (acc[2] + acc[3]))
            return 0

        lax.fori_loop(0, CHUNK // 16, group_body, 0)
        pltpu.sync_copy(dots_v, out_hbm.at[pl.ds(base_w + c * CHUNK, CHUNK)])

    start(0, 0, sem_a, sem_ua)

    def chunk_body(c2, _):
        c = c2 * 2

        @pl.when(c + 1 < N_CHUNKS)
        def _():
            start(c + 1, 1, sem_b, sem_ub)

        wait(0, sem_a, sem_ua)
        compute(c, 0)

        @pl.when(c + 2 < N_CHUNKS)
        def _():
            start(c + 2, 0, sem_a, sem_ua)

        @pl.when(c + 1 < N_CHUNKS)
        def _():
            wait(1, sem_b, sem_ub)
            compute(c + 1, 1)

        return 0

    lax.fori_loop(0, (N_CHUNKS + 1) // 2, chunk_body, 0)


_sc_v_dots = functools.partial(
    pl.kernel,
    mesh=_MESH,
    out_type=jax.ShapeDtypeStruct((TOTAL,), jnp.float32),
    scratch_types=[
        pltpu.VMEM((PER_W,), jnp.int32),
        pltpu.VMEM((PER_W,), jnp.int32),
        pltpu.VMEM((PER_W,), jnp.int32),
        pltpu.VMEM((2 * CHUNK, PAIR_W), jnp.float32),
        pltpu.VMEM((2 * CHUNK, PAIR_W), jnp.float32),
        pltpu.VMEM((CHUNK,), jnp.float32),
        pltpu.SemaphoreType.DMA,
        pltpu.SemaphoreType.DMA,
        pltpu.SemaphoreType.DMA,
        pltpu.SemaphoreType.DMA,
    ],
    compiler_params=_SC_PARAMS,
)(_sc_v_dots_body)


_ROWS = TOTAL // 128
_POS_ROWS = BATCH // 128


def _tc_loss_body(dots_ref, out_ref):
    x = dots_ref[...]
    row = lax.broadcasted_iota(jnp.int32, (_ROWS, 128), 0)
    sgn = jnp.where(row < _POS_ROWS, 1.0, -1.0).astype(jnp.float32)
    z = x * sgn
    ls = jnp.minimum(z, 0.0) - jnp.log1p(jnp.exp(-jnp.abs(z)))
    out_ref[0, 0] = -jnp.sum(ls)


def kernel(u_pos, v_pos, u_neg, v_neg, u_table, v_table):
    u_idx = jnp.concatenate(
        [u_pos.astype(jnp.int32), u_neg.reshape(-1).astype(jnp.int32)])
    v_idx = jnp.concatenate(
        [v_pos.astype(jnp.int32), v_neg.reshape(-1).astype(jnp.int32)])
    u_pairs = u_table.reshape(N_TOKENS // 2, PAIR_W)
    v_pairs = v_table.reshape(N_TOKENS // 2, PAIR_W)
    u_gath = _sc_gather_u(u_idx, u_pairs)
    dots = _sc_v_dots(v_idx, u_idx, v_pairs, u_gath)
    loss = pl.pallas_call(
        _tc_loss_body,
        out_shape=jax.ShapeDtypeStruct((1, 1), jnp.float32),
        out_specs=pl.BlockSpec(memory_space=pltpu.SMEM),
    )(dots.reshape(_ROWS, 128))
    return loss[0, 0]


# speedup vs baseline: 1.0190x; 1.0021x over previous
import functools

import jax
import jax.numpy as jnp
from jax import lax
from jax.experimental import pallas as pl
from jax.experimental.pallas import tpu as pltpu
from jax.experimental.pallas import tpu_sc as plsc

N_TOKENS = 1000000
N_HIDDEN = 64
PAIR_W = 128
BATCH = 16384
N_NEG = 5
TOTAL = BATCH * (1 + N_NEG)
NC = 2
NS = 16
NW = NC * NS
PER_W = TOTAL // NW
CHUNK = 128
N_CHUNKS = PER_W // CHUNK

_SC_PARAMS = pltpu.CompilerParams(
    needs_layout_passes=False, use_tc_tiling_on_sc=True)
_MESH = plsc.VectorSubcoreMesh(core_axis_name="c", subcore_axis_name="s")


def _halve_indices(idx_raw, idx_row):
    def body(i, _):
        v = idx_raw[pl.ds(i * 16, 16)]
        idx_row[pl.ds(i * 16, 16)] = lax.shift_right_logical(v, 1)
        return 0
    lax.fori_loop(0, PER_W // 16, body, 0)


def _sc_gather_u_body(u_idx_hbm, u_table_hbm, out_hbm, idx_raw, idx_row,
                      rows, sem_a, sem_b):
    wid = lax.axis_index("s") * NC + lax.axis_index("c")
    base_w = wid * PER_W
    pltpu.sync_copy(u_idx_hbm.at[pl.ds(base_w, PER_W)], idx_raw)
    _halve_indices(idx_raw, idx_row)

    def start(c, par, sem):
        pltpu.async_copy(
            u_table_hbm.at[idx_row.at[pl.ds(c * CHUNK, CHUNK)]],
            rows.at[pl.ds(par * CHUNK, CHUNK)], sem)

    def wait(par, sem):
        pltpu.make_async_copy(
            u_table_hbm.at[idx_row.at[pl.ds(0, CHUNK)]],
            rows.at[pl.ds(par * CHUNK, CHUNK)], sem).wait()

    def writeback(c, par):
        pltpu.sync_copy(
            rows.at[pl.ds(par * CHUNK, CHUNK)],
            out_hbm.at[pl.ds(base_w + c * CHUNK, CHUNK)])

    start(0, 0, sem_a)

    def chunk_body(c2, _):
        c = c2 * 2

        @pl.when(c + 1 < N_CHUNKS)
        def _():
            start(c + 1, 1, sem_b)

        wait(0, sem_a)
        writeback(c, 0)

        @pl.when(c + 2 < N_CHUNKS)
        def _():
            start(c + 2, 0, sem_a)

        @pl.when(c + 1 < N_CHUNKS)
        def _():
            wait(1, sem_b)
            writeback(c + 1, 1)

        return 0

    lax.fori_loop(0, (N_CHUNKS + 1) // 2, chunk_body, 0)


_sc_gather_u = functools.partial(
    pl.kernel,
    mesh=_MESH,
    out_type=jax.ShapeDtypeStruct((TOTAL, PAIR_W), jnp.float32),
    scratch_types=[
        pltpu.VMEM((PER_W,), jnp.int32),
        pltpu.VMEM((PER_W,), jnp.int32),
        pltpu.VMEM((2 * CHUNK, PAIR_W), jnp.float32),
        pltpu.SemaphoreType.DMA,
        pltpu.SemaphoreType.DMA,
    ],
    compiler_params=_SC_PARAMS,
)(_sc_gather_u_body)


def _sc_v_dots_body(v_idx_hbm, u_idx_hbm, v_table_hbm, u_gath_hbm, out_hbm,
                    idx_raw, idx_row, idx_u, v_rows, u_rows, dots_v,
                    sem_a, sem_b, sem_ua, sem_ub):
    wid = lax.axis_index("s") * NC + lax.axis_index("c")
    base_w = wid * PER_W
    lane = lax.iota(jnp.int32, 16)
    pltpu.sync_copy(v_idx_hbm.at[pl.ds(base_w, PER_W)], idx_raw)
    pltpu.sync_copy(u_idx_hbm.at[pl.ds(base_w, PER_W)], idx_u)
    _halve_indices(idx_raw, idx_row)

    def start(c, par, sem, sem_u):
        pltpu.async_copy(
            v_table_hbm.at[idx_row.at[pl.ds(c * CHUNK, CHUNK)]],
            v_rows.at[pl.ds(par * CHUNK, CHUNK)], sem)
        pltpu.async_copy(
            u_gath_hbm.at[pl.ds(base_w + c * CHUNK, CHUNK)],
            u_rows.at[pl.ds(par * CHUNK, CHUNK)], sem_u)

    def wait(par, sem, sem_u):
        pltpu.make_async_copy(
            v_table_hbm.at[idx_row.at[pl.ds(0, CHUNK)]],
            v_rows.at[pl.ds(par * CHUNK, CHUNK)], sem).wait()
        pltpu.make_async_copy(
            u_gath_hbm.at[pl.ds(base_w, CHUNK)],
            u_rows.at[pl.ds(par * CHUNK, CHUNK)], sem_u).wait()

    def compute(c, par):
        def group_body(g, _):
            rows = par * CHUNK + g * 16 + lane
            off_u = (idx_u[pl.ds(c * CHUNK + g * 16, 16)] & 1) * N_HIDDEN
            off_v = (idx_raw[pl.ds(c * CHUNK + g * 16, 16)] & 1) * N_HIDDEN
            acc = [jnp.zeros((16,), jnp.float32) for _ in range(4)]
            for d in range(N_HIDDEN):
                uu = plsc.load_gather(u_rows, [rows, off_u + d])
                vv = plsc.load_gather(v_rows, [rows, off_v + d])
                acc[d % 4] = acc[d % 4] + uu * vv
            dots_v[pl.ds(g * 16, 16)] = (
                (acc[0] + acc[1]) + (acc[2] + acc[3]))
            return 0

        lax.fori_loop(0, CHUNK // 16, group_body, 0)
        pltpu.sync_copy(dots_v, out_hbm.at[pl.ds(base_w + c * CHUNK, CHUNK)])

    start(0, 0, sem_a, sem_ua)

    def chunk_body(c2, _):
        c = c2 * 2

        @pl.when(c + 1 < N_CHUNKS)
        def _():
            start(c + 1, 1, sem_b, sem_ub)

        wait(0, sem_a, sem_ua)
        compute(c, 0)

        @pl.when(c + 2 < N_CHUNKS)
        def _():
            start(c + 2, 0, sem_a, sem_ua)

        @pl.when(c + 1 < N_CHUNKS)
        def _():
            wait(1, sem_b, sem_ub)
            compute(c + 1, 1)

        return 0

    lax.fori_loop(0, (N_CHUNKS + 1) // 2, chunk_body, 0)


_sc_v_dots = functools.partial(
    pl.kernel,
    mesh=_MESH,
    out_type=jax.ShapeDtypeStruct((TOTAL,), jnp.float32),
    scratch_types=[
        pltpu.VMEM((PER_W,), jnp.int32),
        pltpu.VMEM((PER_W,), jnp.int32),
        pltpu.VMEM((PER_W,), jnp.int32),
        pltpu.VMEM((2 * CHUNK, PAIR_W), jnp.float32),
        pltpu.VMEM((2 * CHUNK, PAIR_W), jnp.float32),
        pltpu.VMEM((CHUNK,), jnp.float32),
        pltpu.SemaphoreType.DMA,
        pltpu.SemaphoreType.DMA,
        pltpu.SemaphoreType.DMA,
        pltpu.SemaphoreType.DMA,
    ],
    compiler_params=_SC_PARAMS,
)(_sc_v_dots_body)


_ROWS = TOTAL // 128
_POS_ROWS = BATCH // 128


def _tc_loss_body(dots_ref, out_ref):
    x = dots_ref[...]
    row = lax.broadcasted_iota(jnp.int32, (_ROWS, 128), 0)
    sgn = jnp.where(row < _POS_ROWS, 1.0, -1.0).astype(jnp.float32)
    z = x * sgn
    ls = jnp.minimum(z, 0.0) - jnp.log1p(jnp.exp(-jnp.abs(z)))
    out_ref[0, 0] = -jnp.sum(ls)


def kernel(u_pos, v_pos, u_neg, v_neg, u_table, v_table):
    u_idx = jnp.concatenate(
        [u_pos.astype(jnp.int32), u_neg.reshape(-1).astype(jnp.int32)])
    v_idx = jnp.concatenate(
        [v_pos.astype(jnp.int32), v_neg.reshape(-1).astype(jnp.int32)])
    u_pairs = u_table.reshape(N_TOKENS // 2, PAIR_W)
    v_pairs = v_table.reshape(N_TOKENS // 2, PAIR_W)
    u_gath = _sc_gather_u(u_idx, u_pairs)
    dots = _sc_v_dots(v_idx, u_idx, v_pairs, u_gath)
    loss = pl.pallas_call(
        _tc_loss_body,
        out_shape=jax.ShapeDtypeStruct((1, 1), jnp.float32),
        out_specs=pl.BlockSpec(memory_space=pltpu.SMEM),
    )(dots.reshape(_ROWS, 128))
    return loss[0, 0]


# speedup vs baseline: 1.1314x; 1.1103x over previous
import functools

import jax
import jax.numpy as jnp
from jax import lax
from jax.experimental import pallas as pl
from jax.experimental.pallas import tpu as pltpu
from jax.experimental.pallas import tpu_sc as plsc

N_TOKENS = 1000000
N_HIDDEN = 64
BATCH = 16384
N_NEG = 5
TOTAL = BATCH * (1 + N_NEG)
NC = 2
NS = 16
NW = NC * NS
PER_W = TOTAL // NW
CHUNK = 256
N_CHUNKS = PER_W // CHUNK

_SC_PARAMS = pltpu.CompilerParams(
    needs_layout_passes=False, use_tc_tiling_on_sc=False)
_MESH = plsc.VectorSubcoreMesh(core_axis_name="c", subcore_axis_name="s")


def _sc_gather_body(idx_hbm, table_hbm, out_hbm, idx_raw, rows, sem_a, sem_b):
    wid = lax.axis_index("s") * NC + lax.axis_index("c")
    base_w = wid * PER_W
    pltpu.sync_copy(idx_hbm.at[pl.ds(base_w, PER_W)], idx_raw)

    def start(c, par, sem):
        pltpu.async_copy(
            table_hbm.at[idx_raw.at[pl.ds(c * CHUNK, CHUNK)]],
            rows.at[pl.ds(par * CHUNK, CHUNK)], sem)

    def wait(par, sem):
        pltpu.make_async_copy(
            table_hbm.at[idx_raw.at[pl.ds(0, CHUNK)]],
            rows.at[pl.ds(par * CHUNK, CHUNK)], sem).wait()

    def writeback(c, par):
        pltpu.sync_copy(
            rows.at[pl.ds(par * CHUNK, CHUNK)],
            out_hbm.at[pl.ds(base_w + c * CHUNK, CHUNK)])

    start(0, 0, sem_a)

    def chunk_body(c2, _):
        c = c2 * 2

        @pl.when(c + 1 < N_CHUNKS)
        def _():
            start(c + 1, 1, sem_b)

        wait(0, sem_a)
        writeback(c, 0)

        @pl.when(c + 2 < N_CHUNKS)
        def _():
            start(c + 2, 0, sem_a)

        @pl.when(c + 1 < N_CHUNKS)
        def _():
            wait(1, sem_b)
            writeback(c + 1, 1)

        return 0

    lax.fori_loop(0, (N_CHUNKS + 1) // 2, chunk_body, 0)


_sc_gather = functools.partial(
    pl.kernel,
    mesh=_MESH,
    out_type=jax.ShapeDtypeStruct((TOTAL, N_HIDDEN), jnp.float32),
    scratch_types=[
        pltpu.VMEM((PER_W,), jnp.int32),
        pltpu.VMEM((2 * CHUNK, N_HIDDEN), jnp.float32),
        pltpu.SemaphoreType.DMA,
        pltpu.SemaphoreType.DMA,
    ],
    compiler_params=_SC_PARAMS,
)(_sc_gather_body)


_VROWS = TOTAL // 2
_POS_VROWS = BATCH // 2
_R_BLK = 8192
_GRID = _VROWS // _R_BLK


def _tc_loss_body(u_ref, v_ref, out_ref):
    i = pl.program_id(0)

    @pl.when(i == 0)
    def _():
        out_ref[0, 0] = 0.0

    prod = u_ref[...] * v_ref[...]
    zl = jnp.sum(prod[:, :N_HIDDEN], axis=1, keepdims=True)
    zr = jnp.sum(prod[:, N_HIDDEN:], axis=1, keepdims=True)
    sgn = jnp.where(i == 0, 1.0, -1.0)

    def neg_log_sigmoid(z):
        zz = z * sgn
        return jnp.log1p(jnp.exp(-jnp.abs(zz))) - jnp.minimum(zz, 0.0)

    out_ref[0, 0] += jnp.sum(neg_log_sigmoid(zl) + neg_log_sigmoid(zr))


def kernel(u_pos, v_pos, u_neg, v_neg, u_table, v_table):
    u_idx = jnp.concatenate(
        [u_pos.astype(jnp.int32), u_neg.reshape(-1).astype(jnp.int32)])
    v_idx = jnp.concatenate(
        [v_pos.astype(jnp.int32), v_neg.reshape(-1).astype(jnp.int32)])
    u_rows = _sc_gather(u_idx, u_table)
    v_rows = _sc_gather(v_idx, v_table)
    loss = pl.pallas_call(
        _tc_loss_body,
        grid=(_GRID,),
        in_specs=[
            pl.BlockSpec((_R_BLK, 2 * N_HIDDEN), lambda i: (i, 0)),
            pl.BlockSpec((_R_BLK, 2 * N_HIDDEN), lambda i: (i, 0)),
        ],
        out_specs=pl.BlockSpec(memory_space=pltpu.SMEM),
        out_shape=jax.ShapeDtypeStruct((1, 1), jnp.float32),
    )(u_rows.reshape(_VROWS, 2 * N_HIDDEN),
      v_rows.reshape(_VROWS, 2 * N_HIDDEN))
    return loss[0, 0]


# speedup vs baseline: 1.1336x; 1.0019x over previous
import functools

import jax
import jax.numpy as jnp
from jax import lax
from jax.experimental import pallas as pl
from jax.experimental.pallas import tpu as pltpu
from jax.experimental.pallas import tpu_sc as plsc

N_TOKENS = 1000000
N_HIDDEN = 64
BATCH = 16384
N_NEG = 5
TOTAL = BATCH * (1 + N_NEG)
NC = 2
NS = 16
NW = NC * NS
PER_W = TOTAL // NW
CHUNK = 256
N_CHUNKS = PER_W // CHUNK

_SC_PARAMS = pltpu.CompilerParams(
    needs_layout_passes=True, use_tc_tiling_on_sc=False)
_MESH = plsc.VectorSubcoreMesh(core_axis_name="c", subcore_axis_name="s")


def _sc_gather_body(idx_hbm, table_hbm, out_hbm, idx_raw, rows, sem_a, sem_b):
    wid = lax.axis_index("s") * NC + lax.axis_index("c")
    base_w = wid * PER_W
    pltpu.sync_copy(idx_hbm.at[pl.ds(base_w, PER_W)], idx_raw)

    def start(c, par, sem):
        pltpu.async_copy(
            table_hbm.at[idx_raw.at[pl.ds(c * CHUNK, CHUNK)]],
            rows.at[pl.ds(par * CHUNK, CHUNK)], sem)

    def wait(par, sem):
        pltpu.make_async_copy(
            table_hbm.at[idx_raw.at[pl.ds(0, CHUNK)]],
            rows.at[pl.ds(par * CHUNK, CHUNK)], sem).wait()

    def writeback(c, par):
        pltpu.sync_copy(
            rows.at[pl.ds(par * CHUNK, CHUNK)],
            out_hbm.at[pl.ds(base_w + c * CHUNK, CHUNK)])

    start(0, 0, sem_a)

    def chunk_body(c2, _):
        c = c2 * 2

        @pl.when(c + 1 < N_CHUNKS)
        def _():
            start(c + 1, 1, sem_b)

        wait(0, sem_a)
        writeback(c, 0)

        @pl.when(c + 2 < N_CHUNKS)
        def _():
            start(c + 2, 0, sem_a)

        @pl.when(c + 1 < N_CHUNKS)
        def _():
            wait(1, sem_b)
            writeback(c + 1, 1)

        return 0

    lax.fori_loop(0, (N_CHUNKS + 1) // 2, chunk_body, 0)


_sc_gather = functools.partial(
    pl.kernel,
    mesh=_MESH,
    out_type=jax.ShapeDtypeStruct((TOTAL, N_HIDDEN), jnp.float32),
    scratch_types=[
        pltpu.VMEM((PER_W,), jnp.int32),
        pltpu.VMEM((2 * CHUNK, N_HIDDEN), jnp.float32),
        pltpu.SemaphoreType.DMA,
        pltpu.SemaphoreType.DMA,
    ],
    compiler_params=_SC_PARAMS,
)(_sc_gather_body)


_VROWS = TOTAL // 2
_POS_VROWS = BATCH // 2
_R_BLK = 8192
_GRID = _VROWS // _R_BLK


def _tc_loss_body(u_ref, v_ref, out_ref):
    i = pl.program_id(0)

    @pl.when(i == 0)
    def _():
        out_ref[0, 0] = 0.0

    prod = u_ref[...] * v_ref[...]
    zl = jnp.sum(prod[:, :N_HIDDEN], axis=1, keepdims=True)
    zr = jnp.sum(prod[:, N_HIDDEN:], axis=1, keepdims=True)
    sgn = jnp.where(i == 0, 1.0, -1.0)

    def neg_log_sigmoid(z):
        zz = z * sgn
        return jnp.log1p(jnp.exp(-jnp.abs(zz))) - jnp.minimum(zz, 0.0)

    out_ref[0, 0] += jnp.sum(neg_log_sigmoid(zl) + neg_log_sigmoid(zr))


def kernel(u_pos, v_pos, u_neg, v_neg, u_table, v_table):
    u_idx = jnp.concatenate(
        [u_pos.astype(jnp.int32), u_neg.reshape(-1).astype(jnp.int32)])
    v_idx = jnp.concatenate(
        [v_pos.astype(jnp.int32), v_neg.reshape(-1).astype(jnp.int32)])
    u_rows = _sc_gather(u_idx, u_table)
    v_rows = _sc_gather(v_idx, v_table)
    loss = pl.pallas_call(
        _tc_loss_body,
        grid=(_GRID,),
        in_specs=[
            pl.BlockSpec((_R_BLK, 2 * N_HIDDEN), lambda i: (i, 0)),
            pl.BlockSpec((_R_BLK, 2 * N_HIDDEN), lambda i: (i, 0)),
        ],
        out_specs=pl.BlockSpec(memory_space=pltpu.SMEM),
        out_shape=jax.ShapeDtypeStruct((1, 1), jnp.float32),
    )(u_rows.reshape(_VROWS, 2 * N_HIDDEN),
      v_rows.reshape(_VROWS, 2 * N_HIDDEN))
    return loss[0, 0]


# speedup vs baseline: 1.1833x; 1.0439x over previous
import functools

import jax
import jax.numpy as jnp
from jax import lax
from jax.experimental import pallas as pl
from jax.experimental.pallas import tpu as pltpu
from jax.experimental.pallas import tpu_sc as plsc

N_TOKENS = 1000000
N_HIDDEN = 64
BATCH = 16384
N_NEG = 5
TOTAL = BATCH * (1 + N_NEG)
NC = 2
NS = 16
NW = NC * NS
PER_W = TOTAL // NW
CHUNK = 256
N_CHUNKS = PER_W // CHUNK

_SC_PARAMS = pltpu.CompilerParams(
    needs_layout_passes=False, use_tc_tiling_on_sc=False)
_MESH = plsc.VectorSubcoreMesh(core_axis_name="c", subcore_axis_name="s")


def _sc_dots_body(u_idx_hbm, v_idx_hbm, u_table_hbm, v_table_hbm, out_hbm,
                  u_idx, v_idx, u_rows, v_rows, dots_v,
                  sem_ua, sem_ub, sem_va, sem_vb):
    wid = lax.axis_index("s") * NC + lax.axis_index("c")
    base_w = wid * PER_W
    pltpu.sync_copy(u_idx_hbm.at[pl.ds(base_w, PER_W)], u_idx)
    pltpu.sync_copy(v_idx_hbm.at[pl.ds(base_w, PER_W)], v_idx)

    def start(c, par, sem_u, sem_v):
        pltpu.async_copy(
            u_table_hbm.at[u_idx.at[pl.ds(c * CHUNK, CHUNK)]],
            u_rows.at[pl.ds(par * CHUNK, CHUNK)], sem_u)
        pltpu.async_copy(
            v_table_hbm.at[v_idx.at[pl.ds(c * CHUNK, CHUNK)]],
            v_rows.at[pl.ds(par * CHUNK, CHUNK)], sem_v)

    def wait(par, sem_u, sem_v):
        pltpu.make_async_copy(
            u_table_hbm.at[u_idx.at[pl.ds(0, CHUNK)]],
            u_rows.at[pl.ds(par * CHUNK, CHUNK)], sem_u).wait()
        pltpu.make_async_copy(
            v_table_hbm.at[v_idx.at[pl.ds(0, CHUNK)]],
            v_rows.at[pl.ds(par * CHUNK, CHUNK)], sem_v).wait()

    lane = lax.iota(jnp.int32, 16)

    def compute(c, par):
        def group_body(g, _):
            vec = jnp.zeros((16,), jnp.float32)
            for j in range(16):
                r = par * CHUNK + g * 16 + j
                acc = (u_rows[r, pl.ds(0, 16)] * v_rows[r, pl.ds(0, 16)]
                       + u_rows[r, pl.ds(16, 16)] * v_rows[r, pl.ds(16, 16)]
                       ) + (
                      u_rows[r, pl.ds(32, 16)] * v_rows[r, pl.ds(32, 16)]
                       + u_rows[r, pl.ds(48, 16)] * v_rows[r, pl.ds(48, 16)])
                vec = jnp.where(lane == j, jnp.sum(acc), vec)
            dots_v[pl.ds(g * 16, 16)] = vec
            return 0

        lax.fori_loop(0, CHUNK // 16, group_body, 0)
        pltpu.sync_copy(dots_v, out_hbm.at[pl.ds(base_w + c * CHUNK, CHUNK)])

    start(0, 0, sem_ua, sem_va)

    def chunk_body(c2, _):
        c = c2 * 2

        @pl.when(c + 1 < N_CHUNKS)
        def _():
            start(c + 1, 1, sem_ub, sem_vb)

        wait(0, sem_ua, sem_va)
        compute(c, 0)

        @pl.when(c + 2 < N_CHUNKS)
        def _():
            start(c + 2, 0, sem_ua, sem_va)

        @pl.when(c + 1 < N_CHUNKS)
        def _():
            wait(1, sem_ub, sem_vb)
            compute(c + 1, 1)

        return 0

    lax.fori_loop(0, (N_CHUNKS + 1) // 2, chunk_body, 0)


_sc_dots = functools.partial(
    pl.kernel,
    mesh=_MESH,
    out_type=jax.ShapeDtypeStruct((TOTAL,), jnp.float32),
    scratch_types=[
        pltpu.VMEM((PER_W,), jnp.int32),
        pltpu.VMEM((PER_W,), jnp.int32),
        pltpu.VMEM((2 * CHUNK, N_HIDDEN), jnp.float32),
        pltpu.VMEM((2 * CHUNK, N_HIDDEN), jnp.float32),
        pltpu.VMEM((CHUNK,), jnp.float32),
        pltpu.SemaphoreType.DMA,
        pltpu.SemaphoreType.DMA,
        pltpu.SemaphoreType.DMA,
        pltpu.SemaphoreType.DMA,
    ],
    compiler_params=_SC_PARAMS,
)(_sc_dots_body)


_ROWS = TOTAL // 128
_POS_ROWS = BATCH // 128


def _tc_loss_body(dots_ref, out_ref):
    x = dots_ref[...]
    row = lax.broadcasted_iota(jnp.int32, (_ROWS, 128), 0)
    sgn = jnp.where(row < _POS_ROWS, 1.0, -1.0).astype(jnp.float32)
    z = x * sgn
    ls = jnp.minimum(z, 0.0) - jnp.log1p(jnp.exp(-jnp.abs(z)))
    out_ref[0, 0] = -jnp.sum(ls)


def kernel(u_pos, v_pos, u_neg, v_neg, u_table, v_table):
    u_idx = jnp.concatenate(
        [u_pos.astype(jnp.int32), u_neg.reshape(-1).astype(jnp.int32)])
    v_idx = jnp.concatenate(
        [v_pos.astype(jnp.int32), v_neg.reshape(-1).astype(jnp.int32)])
    dots = _sc_dots(u_idx, v_idx, u_table, v_table)
    loss = pl.pallas_call(
        _tc_loss_body,
        out_shape=jax.ShapeDtypeStruct((1, 1), jnp.float32),
        out_specs=pl.BlockSpec(memory_space=pltpu.SMEM),
    )(dots.reshape(_ROWS, 128))
    return loss[0, 0]
